# baseline (speedup 1.0000x reference)
import functools

import jax
import jax.numpy as jnp
from jax import lax
from jax.experimental import pallas as pl
from jax.experimental.pallas import tpu as pltpu
from jax.experimental.pallas import tpu_sc as plsc

N_FEAT = 7
ROWS = 129
EMBED_DIM = 64
PADD = 128
HIDDEN = 1024
N_TOK = 16384
TROWS = N_FEAT * ROWS
TROWS_PAD = 904

NC, NS, L = 2, 16, 16
NW = NC * NS
TOK_PER_W = N_TOK // NW
ROWS_PER_W = TOK_PER_W * N_FEAT
GTOK = 32
GROWS = GTOK * N_FEAT
NGROUP = TOK_PER_W // GTOK
CHUNK = 112
NCHUNK = GROWS // CHUNK

_sc_mesh = plsc.VectorSubcoreMesh(
    core_axis_name="c", subcore_axis_name="s", num_cores=NC, num_subcores=NS)


@functools.partial(
    pl.kernel,
    out_type=jax.ShapeDtypeStruct((N_TOK * N_FEAT, PADD), jnp.float32),
    mesh=_sc_mesh,
    scratch_types=[
        pltpu.VMEM((ROWS_PER_W,), jnp.int32),
        pltpu.VMEM((NGROUP * NCHUNK, CHUNK), jnp.int32),
        pltpu.VMEM((CHUNK,), jnp.int32),
        pltpu.VMEM((2, GROWS, PADD), jnp.float32),
        pltpu.VMEM_SHARED((TROWS_PAD, PADD), jnp.float32),
        pltpu.SemaphoreType.DMA,
        pltpu.SemaphoreType.DMA,
    ],
)
def _sc_gather(ids_hbm, table_hbm, offs_hbm, out_hbm, ids_v, idx_v, offs_v,
               bufs, table_sp, sem0, sem1):
    wid = lax.axis_index("s") * NC + lax.axis_index("c")
    row0 = wid * ROWS_PER_W
    @pl.when(lax.axis_index("s") == 0)
    def _stage():
        pltpu.sync_copy(table_hbm, table_sp)
    pltpu.sync_copy(ids_hbm.at[pl.ds(row0, ROWS_PER_W)], ids_v)
    pltpu.sync_copy(offs_hbm, offs_v)
    for j in range(NGROUP * NCHUNK):
        for k in range(CHUNK // L):
            idx_v[j, pl.ds(k * L, L)] = (
                ids_v[pl.ds(j * CHUNK + k * L, L)] + offs_v[pl.ds(k * L, L)])
    plsc.subcore_barrier()

    sems = (sem0, sem1)

    def issue(g):
        buf = bufs.at[g % 2]
        handles = []
        for q in range(NCHUNK):
            handles.append(pltpu.async_copy(
                table_sp.at[idx_v.at[g * NCHUNK + q]],
                buf.at[pl.ds(q * CHUNK, CHUNK)],
                sems[g % 2]))
        return handles

    pending = {0: issue(0)}
    for g in range(NGROUP):
        if g + 1 < NGROUP:
            pending[g + 1] = issue(g + 1)
        for h in pending.pop(g):
            h.wait()
        pltpu.sync_copy(bufs.at[g % 2],
                        out_hbm.at[pl.ds(row0 + g * GROWS, GROWS)])


TOK_BLOCK = 2048


def _mm_body(x_ref, w_ref, b_ref, out_ref):
    out_ref[...] = jnp.dot(x_ref[...], w_ref[...],
                           preferred_element_type=jnp.float32) + b_ref[...]


def _matmul(x, W_pad, b2d):
    grid = (N_TOK // TOK_BLOCK,)
    return pl.pallas_call(
        _mm_body,
        grid=grid,
        in_specs=[
            pl.BlockSpec((TOK_BLOCK, N_FEAT * PADD), lambda t: (t, 0)),
            pl.BlockSpec((N_FEAT * PADD, HIDDEN), lambda t: (0, 0)),
            pl.BlockSpec((1, HIDDEN), lambda t: (0, 0)),
        ],
        out_specs=pl.BlockSpec((TOK_BLOCK, HIDDEN), lambda t: (t, 0)),
        out_shape=jax.ShapeDtypeStruct((N_TOK, HIDDEN), jnp.float32),
    )(x, W_pad, b2d)


@jax.jit
def _run(ids_flat, table_pad, offs, W_pad, b2d):
    rows = _sc_gather(ids_flat, table_pad, offs)
    x = rows.reshape(N_TOK, N_FEAT * PADD)
    return _matmul(x, W_pad, b2d)


def kernel(noise_ids, emb, W, b):
    B, S, F = noise_ids.shape
    ids_flat = jnp.clip(noise_ids, 0, ROWS - 1).astype(jnp.int32).reshape(B * S * F)
    table = emb.reshape(TROWS, EMBED_DIM)
    table_pad = jnp.pad(table, ((0, TROWS_PAD - TROWS), (0, PADD - EMBED_DIM)))
    offs = ((jnp.arange(CHUNK, dtype=jnp.int32) % N_FEAT) * ROWS)
    W_pad = jnp.pad(W.reshape(N_FEAT, EMBED_DIM, HIDDEN),
                    ((0, 0), (0, PADD - EMBED_DIM), (0, 0))).reshape(N_FEAT * PADD, HIDDEN)
    out = _run(ids_flat, table_pad, offs, W_pad, b[None, :])
    return out.reshape(B, S, HIDDEN)


# speedup vs baseline: 1.3927x; 1.3927x over previous
import functools

import jax
import jax.numpy as jnp
from jax import lax
from jax.experimental import pallas as pl
from jax.experimental.pallas import tpu as pltpu
from jax.experimental.pallas import tpu_sc as plsc

N_FEAT = 7
ROWS = 129
EMBED_DIM = 64
PADD = 128
HIDDEN = 1024
N_TOK = 16384
TROWS = N_FEAT * ROWS
TROWS_PAD = 904

NC, NS, L = 2, 16, 16
NW = NC * NS
TOK_PER_W = N_TOK // NW
ROWS_PER_W = TOK_PER_W * N_FEAT
GTOK = 32
GROWS = GTOK * N_FEAT
NGROUP = TOK_PER_W // GTOK
CHUNK = 112
NCHUNK = GROWS // CHUNK

_sc_mesh = plsc.VectorSubcoreMesh(
    core_axis_name="c", subcore_axis_name="s", num_cores=NC, num_subcores=NS)


@functools.partial(
    pl.kernel,
    out_type=jax.ShapeDtypeStruct((N_TOK * N_FEAT, PADD), jnp.float32),
    mesh=_sc_mesh,
    scratch_types=[
        pltpu.VMEM((ROWS_PER_W,), jnp.int32),
        pltpu.VMEM((NGROUP * NCHUNK, CHUNK), jnp.int32),
        pltpu.VMEM((CHUNK,), jnp.int32),
        pltpu.VMEM((2, GROWS, PADD), jnp.float32),
        pltpu.VMEM_SHARED((TROWS_PAD, PADD), jnp.float32),
        pltpu.SemaphoreType.DMA,
        pltpu.SemaphoreType.DMA,
    ],
)
def _sc_gather(ids_hbm, table_hbm, offs_hbm, out_hbm, ids_v, idx_v, offs_v,
               bufs, table_sp, sem0, sem1):
    wid = lax.axis_index("s") * NC + lax.axis_index("c")
    row0 = wid * ROWS_PER_W
    @pl.when(lax.axis_index("s") == 0)
    def _stage():
        pltpu.sync_copy(table_hbm, table_sp)
    pltpu.sync_copy(ids_hbm.at[pl.ds(row0, ROWS_PER_W)], ids_v)
    pltpu.sync_copy(offs_hbm, offs_v)
    for j in range(NGROUP * NCHUNK):
        for k in range(CHUNK // L):
            idx_v[j, pl.ds(k * L, L)] = (
                ids_v[pl.ds(j * CHUNK + k * L, L)] + offs_v[pl.ds(k * L, L)])
    plsc.subcore_barrier()

    sems = (sem0, sem1)

    def issue(g):
        buf = bufs.at[g % 2]
        handles = []
        for q in range(NCHUNK):
            handles.append(pltpu.async_copy(
                table_sp.at[idx_v.at[g * NCHUNK + q]],
                buf.at[pl.ds(q * CHUNK, CHUNK)],
                sems[g % 2]))
        return handles

    pending = {0: issue(0)}
    for g in range(NGROUP):
        if g + 1 < NGROUP:
            pending[g + 1] = issue(g + 1)
        for h in pending.pop(g):
            h.wait()
        pltpu.sync_copy(bufs.at[g % 2],
                        out_hbm.at[pl.ds(row0 + g * GROWS, GROWS)])


TOK_BLOCK = 2048


def _mm_body(x_ref, w_ref, b_ref, out_ref):
    parts = [x_ref[:, i].reshape(TOK_BLOCK, PADD) for i in range(N_FEAT)]
    x = jnp.concatenate(parts, axis=1)
    out_ref[...] = jnp.dot(x, w_ref[...],
                           preferred_element_type=jnp.float32) + b_ref[...]


def _matmul(rows4d, W_pad, b2d):
    grid = (N_TOK // TOK_BLOCK,)
    return pl.pallas_call(
        _mm_body,
        grid=grid,
        in_specs=[
            pl.BlockSpec((TOK_BLOCK // 8, N_FEAT, 8, PADD), lambda t: (t, 0, 0, 0)),
            pl.BlockSpec((N_FEAT * PADD, HIDDEN), lambda t: (0, 0)),
            pl.BlockSpec((1, HIDDEN), lambda t: (0, 0)),
        ],
        out_specs=pl.BlockSpec((TOK_BLOCK, HIDDEN), lambda t: (t, 0)),
        out_shape=jax.ShapeDtypeStruct((N_TOK, HIDDEN), jnp.float32),
    )(rows4d, W_pad, b2d)


@jax.jit
def _run(ids_perm, table_pad, offs, W_pad, b2d):
    rows = _sc_gather(ids_perm, table_pad, offs)
    rows4d = rows.reshape(N_TOK // 8, N_FEAT, 8, PADD)
    return _matmul(rows4d, W_pad, b2d)


def kernel(noise_ids, emb, W, b):
    B, S, F = noise_ids.shape
    ids32 = jnp.clip(noise_ids, 0, ROWS - 1).astype(jnp.int32).reshape(B * S, F)
    ids_perm = ids32.reshape(N_TOK // 8, 8, F).transpose(0, 2, 1).reshape(-1)
    table = emb.reshape(TROWS, EMBED_DIM)
    table_pad = jnp.pad(table, ((0, TROWS_PAD - TROWS), (0, PADD - EMBED_DIM)))
    offs = ((jnp.arange(CHUNK, dtype=jnp.int32) % 56) // 8) * ROWS
    W_pad = jnp.pad(W.reshape(N_FEAT, EMBED_DIM, HIDDEN),
                    ((0, 0), (0, PADD - EMBED_DIM), (0, 0))).reshape(N_FEAT * PADD, HIDDEN)
    out = _run(ids_perm, table_pad, offs, W_pad, b[None, :])
    return out.reshape(B, S, HIDDEN)


# speedup vs baseline: 2.5654x; 1.8420x over previous
import jax
import jax.numpy as jnp
from jax.experimental import pallas as pl

N_FEAT = 7
BINS = 128
EMBED_DIM = 64
HIDDEN = 1024
N_TOK = 16384
KDIM = N_FEAT * BINS

TOK_BLOCK = 2048


def _prep_body(emb_ref, w_ref, t_ref):
    for i in range(N_FEAT):
        t = jnp.dot(emb_ref[i, :BINS, :],
                    w_ref[i * EMBED_DIM:(i + 1) * EMBED_DIM, :],
                    preferred_element_type=jnp.float32)
        t_ref[i * BINS:(i + 1) * BINS, :] = t.astype(jnp.bfloat16)


def _prep(emb, W):
    return pl.pallas_call(
        _prep_body,
        out_shape=jax.ShapeDtypeStruct((KDIM, HIDDEN), jnp.bfloat16),
    )(emb, W)


def _fused_body(ids_ref, t_ref, b_ref, out_ref):
    parts = []
    for i in range(N_FEAT):
        ids_col = ids_ref[:, i][:, None]
        iota = jax.lax.broadcasted_iota(jnp.int32, (TOK_BLOCK, BINS), 1)
        parts.append((ids_col == iota).astype(jnp.bfloat16))
    oh = jnp.concatenate(parts, axis=-1)
    out_ref[...] = (jnp.dot(oh, t_ref[...], preferred_element_type=jnp.float32)
                    + b_ref[...])


@jax.jit
def _run(ids32, emb, W, b2d):
    t_tab = _prep(emb, W)
    grid = (N_TOK // TOK_BLOCK,)
    return pl.pallas_call(
        _fused_body,
        grid=grid,
        in_specs=[
            pl.BlockSpec((TOK_BLOCK, 8), lambda t: (t, 0)),
            pl.BlockSpec((KDIM, HIDDEN), lambda t: (0, 0)),
            pl.BlockSpec((1, HIDDEN), lambda t: (0, 0)),
        ],
        out_specs=pl.BlockSpec((TOK_BLOCK, HIDDEN), lambda t: (t, 0)),
        out_shape=jax.ShapeDtypeStruct((N_TOK, HIDDEN), jnp.float32),
    )(ids32, t_tab, b2d)


def kernel(noise_ids, emb, W, b):
    B, S, F = noise_ids.shape
    ids32 = jnp.clip(noise_ids, 0, BINS - 1).astype(jnp.int32).reshape(B * S, F)
    ids32 = jnp.pad(ids32, ((0, 0), (0, 8 - F)))
    out = _run(ids32, emb, W, b[None, :])
    return out.reshape(B, S, HIDDEN)


# speedup vs baseline: 2.7404x; 1.0682x over previous
import functools

import jax
import jax.numpy as jnp
from jax.experimental import pallas as pl
from jax.experimental.pallas import tpu as pltpu

N_FEAT = 7
ROWS = 129
EMBED_DIM = 64
HIDDEN = 1024

TOK_BLOCK = 1024


def _fused_body(ids_ref, emb_ref, w_ref, b_ref, out_ref):
    parts = []
    for i in range(N_FEAT):
        ids_col = ids_ref[:, i][:, None]
        iota = jax.lax.broadcasted_iota(jnp.int32, (TOK_BLOCK, ROWS), 1)
        oh = (ids_col == iota).astype(jnp.float32)
        tbl = emb_ref[i * ROWS:(i + 1) * ROWS, :]
        parts.append(jnp.dot(oh, tbl, preferred_element_type=jnp.float32))
    x = jnp.concatenate(parts, axis=-1)
    acc = jnp.dot(x, w_ref[...], preferred_element_type=jnp.float32)
    out_ref[...] = acc + b_ref[...]


@jax.jit
def _fused(ids32, emb_flat, W, b):
    n_tok = ids32.shape[0]
    grid = (n_tok // TOK_BLOCK,)
    return pl.pallas_call(
        _fused_body,
        grid=grid,
        in_specs=[
            pl.BlockSpec((TOK_BLOCK, 8), lambda t: (t, 0)),
            pl.BlockSpec((N_FEAT * ROWS, EMBED_DIM), lambda t: (0, 0)),
            pl.BlockSpec((N_FEAT * EMBED_DIM, HIDDEN), lambda t: (0, 0)),
            pl.BlockSpec((1, HIDDEN), lambda t: (0, 0)),
        ],
        out_specs=pl.BlockSpec((TOK_BLOCK, HIDDEN), lambda t: (t, 0)),
        out_shape=jax.ShapeDtypeStruct((n_tok, HIDDEN), jnp.float32),
    )(ids32, emb_flat, W, b)


def kernel(noise_ids, emb, W, b):
    B, S, F = noise_ids.shape
    ids32 = jnp.clip(noise_ids, 0, ROWS - 1).astype(jnp.int32).reshape(B * S, F)
    ids32 = jnp.pad(ids32, ((0, 0), (0, 8 - F)))
    emb_flat = emb.reshape(N_FEAT * ROWS, EMBED_DIM)
    out = _fused(ids32, emb_flat, W, b[None, :])
    return out.reshape(B, S, HIDDEN)


# speedup vs baseline: 2.7477x; 1.0027x over previous
import functools

import jax
import jax.numpy as jnp
from jax.experimental import pallas as pl
from jax.experimental.pallas import tpu as pltpu

N_FEAT = 7
ROWS = 129
EMBED_DIM = 64
HIDDEN = 1024

TOK_BLOCK = 4096


def _fused_body(ids_ref, emb_ref, w_ref, b_ref, out_ref):
    parts = []
    for i in range(N_FEAT):
        ids_col = ids_ref[:, i][:, None]
        iota = jax.lax.broadcasted_iota(jnp.int32, (TOK_BLOCK, ROWS), 1)
        oh = (ids_col == iota).astype(jnp.float32)
        tbl = emb_ref[i * ROWS:(i + 1) * ROWS, :]
        parts.append(jnp.dot(oh, tbl, preferred_element_type=jnp.float32))
    x = jnp.concatenate(parts, axis=-1)
    acc = jnp.dot(x, w_ref[...], preferred_element_type=jnp.float32)
    out_ref[...] = acc + b_ref[...]


@jax.jit
def _fused(ids32, emb_flat, W, b):
    n_tok = ids32.shape[0]
    grid = (n_tok // TOK_BLOCK,)
    return pl.pallas_call(
        _fused_body,
        grid=grid,
        in_specs=[
            pl.BlockSpec((TOK_BLOCK, 8), lambda t: (t, 0)),
            pl.BlockSpec((N_FEAT * ROWS, EMBED_DIM), lambda t: (0, 0)),
            pl.BlockSpec((N_FEAT * EMBED_DIM, HIDDEN), lambda t: (0, 0)),
            pl.BlockSpec((1, HIDDEN), lambda t: (0, 0)),
        ],
        out_specs=pl.BlockSpec((TOK_BLOCK, HIDDEN), lambda t: (t, 0)),
        out_shape=jax.ShapeDtypeStruct((n_tok, HIDDEN), jnp.float32),
    )(ids32, emb_flat, W, b)


def kernel(noise_ids, emb, W, b):
    B, S, F = noise_ids.shape
    ids32 = jnp.clip(noise_ids, 0, ROWS - 1).astype(jnp.int32).reshape(B * S, F)
    ids32 = jnp.pad(ids32, ((0, 0), (0, 8 - F)))
    emb_flat = emb.reshape(N_FEAT * ROWS, EMBED_DIM)
    out = _fused(ids32, emb_flat, W, b[None, :])
    return out.reshape(B, S, HIDDEN)
